# baseline (speedup 1.0000x reference)
import jax
import jax.numpy as jnp
from jax import lax
from jax.experimental import pallas as pl
from jax.experimental.pallas import tpu as pltpu
from jax.experimental.pallas import tpu_sc as plsc

N = 10000
E = 320000
D_IN = 128
D_HID = 128
D_OUT = 64

NC, NS = 2, 16
NW = NC * NS
CHUNK = 128
CHUNKS_PER_TILE = 80
GROUP = 2
NGROUPS = CHUNKS_PER_TILE // GROUP
P0, P1 = 19, 61
E_TILE = CHUNKS_PER_TILE * CHUNK
E_PAD = NW * E_TILE
N_PAD = 10240
ROWS_PER_SUB = N_PAD // NS


def _sc_segsum(d, want_deg):
    mesh = plsc.VectorSubcoreMesh(core_axis_name="c", subcore_axis_name="s")

    def body(x_hbm, idx_hbm, *rest):
        if want_deg:
            agg_hbm, deg_hbm = rest[0], rest[1]
            scr = rest[2:]
        else:
            agg_hbm = rest[0]
            scr = rest[1:]
        idx, rows, ones, acc, dacc, isem, gsem, ssem = scr
        c = lax.axis_index("c")
        s = lax.axis_index("s")
        nloc = jnp.where(c == 0, P0, P1)
        gbase = jnp.where(c == 0, s * P0, NS * P0 + s * P1)

        def load_group(g):
            return pltpu.async_copy(idx_hbm.at[gbase + g], idx.at[g % 2], isem)

        ip = load_group(0)

        def zr(i, _):
            for jj in range(d // 16):
                rows[0, i, pl.ds(jj * 16, 16)] = jnp.zeros((16,), jnp.float32)
            return 0
        lax.fori_loop(0, CHUNK, zr, 0)
        for jj in range(CHUNK // 16):
            ones[pl.ds(jj * 16, 16)] = jnp.ones((16,), jnp.float32)

        def zc(k, _):
            r0 = s * ROWS_PER_SUB + k * CHUNK
            pltpu.sync_copy(rows.at[0], acc.at[pl.ds(r0, CHUNK)])
            if want_deg:
                pltpu.sync_copy(rows.at[0, 0], dacc.at[pl.ds(r0, CHUNK)])
            return 0
        lax.fori_loop(0, ROWS_PER_SUB // CHUNK, zc, 0)
        plsc.subcore_barrier()

        def gathers(par):
            gds = [pltpu.async_copy(x_hbm.at[idx.at[par, b, 0]],
                                    rows.at[b], gsem)
                   for b in range(GROUP)]
            for gd in gds:
                gd.wait()

        def scatters(par):
            for b in range(GROUP):
                pltpu.async_copy(rows.at[b], acc.at[idx.at[par, b, 1]],
                                 ssem, add=True)
                if want_deg:
                    pltpu.async_copy(ones, dacc.at[idx.at[par, b, 1]],
                                     ssem, add=True)

        def drain_scatters(par):
            for b in range(GROUP):
                pltpu.make_async_copy(rows.at[b], acc.at[idx.at[par, b, 1]],
                                      ssem).wait()
                if want_deg:
                    pltpu.make_async_copy(ones, dacc.at[idx.at[par, b, 1]],
                                          ssem).wait()

        ip.wait()
        load_group(1)
        gathers(0)
        scatters(0)
        drain_scatters(0)

        def step(g, carry):
            par = g % 2
            pltpu.make_async_copy(idx_hbm.at[gbase + g], idx.at[par], isem).wait()
            gn = jnp.minimum(g + 1, nloc - 1)
            pltpu.async_copy(idx_hbm.at[gbase + gn], idx.at[(g + 1) % 2], isem)
            gathers(par)
            scatters(par)
            drain_scatters(par)
            return carry

        lax.fori_loop(1, nloc, step, 0)
        pltpu.make_async_copy(idx_hbm.at[gbase + nloc - 1],
                              idx.at[nloc % 2], isem).wait()
        plsc.subcore_barrier()

        wds = []
        for k in range(ROWS_PER_SUB // CHUNK):
            r0 = s * ROWS_PER_SUB + k * CHUNK
            wds.append(pltpu.async_copy(acc.at[pl.ds(r0, CHUNK)],
                                        agg_hbm.at[c, pl.ds(r0, CHUNK)], isem))
        if want_deg:
            wds.append(pltpu.async_copy(dacc.at[pl.ds(s * ROWS_PER_SUB, ROWS_PER_SUB)],
                                        deg_hbm.at[c, pl.ds(s * ROWS_PER_SUB,
                                                            ROWS_PER_SUB)], isem))
        for wd in wds:
            wd.wait()

    out_type = [jax.ShapeDtypeStruct((NC, N_PAD, d), jnp.float32)]
    if want_deg:
        out_type.append(jax.ShapeDtypeStruct((NC, N_PAD), jnp.float32))
    return pl.kernel(
        body,
        out_type=tuple(out_type),
        mesh=mesh,
        scratch_types=[
            pltpu.VMEM((2, GROUP, 2, CHUNK), jnp.int32),
            pltpu.VMEM((GROUP, CHUNK, d), jnp.float32),
            pltpu.VMEM((CHUNK,), jnp.float32),
            pltpu.VMEM_SHARED((N_PAD, d), jnp.float32),
            pltpu.VMEM_SHARED((N_PAD,), jnp.float32),
            pltpu.SemaphoreType.DMA,
            pltpu.SemaphoreType.DMA,
            pltpu.SemaphoreType.DMA,
        ],
    )


RB = 400


def _dense(d_in, d_out, relu):
    def body(aggp_ref, degp_ref, x_ref, wlt_ref, wrt_ref, bl_ref, o_ref):
        agg = aggp_ref[0] + aggp_ref[1]
        deg = degp_ref[0, 0, 0] + degp_ref[1, 0, 0]
        mean = agg / jnp.clip(deg, 1.0, None)[:, None]
        out = (jnp.dot(mean, wlt_ref[...], preferred_element_type=jnp.float32)
               + jnp.dot(x_ref[...], wrt_ref[...], preferred_element_type=jnp.float32)
               + bl_ref[...])
        o_ref[...] = jnp.maximum(out, 0.0) if relu else out

    return pl.pallas_call(
        body,
        grid=(N // RB,),
        in_specs=[
            pl.BlockSpec((NC, RB, d_in), lambda i: (0, i, 0)),
            pl.BlockSpec((NC, 1, 1, RB), lambda i: (0, i, 0, 0)),
            pl.BlockSpec((RB, d_in), lambda i: (i, 0)),
            pl.BlockSpec((d_in, d_out), lambda i: (0, 0)),
            pl.BlockSpec((d_in, d_out), lambda i: (0, 0)),
            pl.BlockSpec((1, d_out), lambda i: (0, 0)),
        ],
        out_specs=pl.BlockSpec((RB, d_out), lambda i: (i, 0)),
        out_shape=jax.ShapeDtypeStruct((N, d_out), jnp.float32),
    )


_segsum_deg = _sc_segsum(D_IN, want_deg=True)
_segsum_nodeg = _sc_segsum(D_IN, want_deg=False)
_dense1 = _dense(D_IN, D_HID, relu=True)
_dense2 = _dense(D_HID, D_OUT, relu=False)


@jax.jit
def kernel(x, edge_index, Wl1, bl1, Wr1, Wl2, bl2, Wr2):
    src = edge_index[0].astype(jnp.int32)
    dst = edge_index[1].astype(jnp.int32)
    pad = E_PAD - E
    src_p = jnp.concatenate([src, jnp.zeros((pad,), jnp.int32)]).reshape(-1, CHUNK)
    dst_p = jnp.concatenate([dst, jnp.full((pad,), N, jnp.int32)]).reshape(-1, CHUNK)
    idx_all = jnp.stack([src_p, dst_p], axis=1).reshape(-1, GROUP, 2, CHUNK)

    agg1, deg = _segsum_deg(x, idx_all)
    deg_r = deg[:, :N].reshape(NC, N // RB, 1, RB)
    h = _dense1(agg1, deg_r, x, Wl1.T, Wr1.T, bl1.reshape(1, -1))
    res2 = _segsum_nodeg(h, idx_all)
    agg2 = res2[0] if isinstance(res2, (tuple, list)) else res2
    return _dense2(agg2, deg_r, h, Wl2.T, Wr2.T, bl2.reshape(1, -1))


# speedup vs baseline: 1.2773x; 1.2773x over previous
import jax
import jax.numpy as jnp
from jax import lax
from jax.experimental import pallas as pl
from jax.experimental.pallas import tpu as pltpu
from jax.experimental.pallas import tpu_sc as plsc

N = 10000
E = 320000
D_IN = 128
D_HID = 128
D_OUT = 64

NC, NS = 2, 16
NW = NC * NS
CHUNK = 128
CHUNKS_PER_TILE = 80
GROUP = 2
NGROUPS = CHUNKS_PER_TILE // GROUP
P0, P1 = 61, 19
E_TILE = CHUNKS_PER_TILE * CHUNK
E_PAD = NW * E_TILE
N_PAD = 10240
ROWS_PER_SUB = N_PAD // NS


def _sc_segsum(d, want_deg):
    mesh = plsc.VectorSubcoreMesh(core_axis_name="c", subcore_axis_name="s")

    def body(x_hbm, idx_hbm, *rest):
        if want_deg:
            agg_hbm, deg_hbm = rest[0], rest[1]
            scr = rest[2:]
        else:
            agg_hbm = rest[0]
            scr = rest[1:]
        idx, rows, ones, acc, dacc, isem, gsem, ssem = scr
        c = lax.axis_index("c")
        s = lax.axis_index("s")
        nloc = jnp.where(c == 0, P0, P1)
        gbase = jnp.where(c == 0, s * P0, NS * P0 + s * P1)

        def load_group(g):
            return pltpu.async_copy(idx_hbm.at[gbase + g], idx.at[g % 2], isem)

        ip = load_group(0)

        def zr(i, _):
            for jj in range(d // 16):
                rows[0, i, pl.ds(jj * 16, 16)] = jnp.zeros((16,), jnp.float32)
            return 0
        lax.fori_loop(0, CHUNK, zr, 0)
        for jj in range(CHUNK // 16):
            ones[pl.ds(jj * 16, 16)] = jnp.ones((16,), jnp.float32)

        def zc(k, _):
            r0 = s * ROWS_PER_SUB + k * CHUNK
            pltpu.sync_copy(rows.at[0], acc.at[pl.ds(r0, CHUNK)])
            if want_deg:
                pltpu.sync_copy(rows.at[0, 0], dacc.at[pl.ds(r0, CHUNK)])
            return 0
        lax.fori_loop(0, ROWS_PER_SUB // CHUNK, zc, 0)
        plsc.subcore_barrier()

        def gathers(par):
            gds = [pltpu.async_copy(x_hbm.at[idx.at[par, b, 0]],
                                    rows.at[b], gsem)
                   for b in range(GROUP)]
            for gd in gds:
                gd.wait()

        def scatters(par):
            for b in range(GROUP):
                pltpu.async_copy(rows.at[b], acc.at[idx.at[par, b, 1]],
                                 ssem, add=True)
                if want_deg:
                    pltpu.async_copy(ones, dacc.at[idx.at[par, b, 1]],
                                     ssem, add=True)

        def drain_scatters(par):
            for b in range(GROUP):
                pltpu.make_async_copy(rows.at[b], acc.at[idx.at[par, b, 1]],
                                      ssem).wait()
                if want_deg:
                    pltpu.make_async_copy(ones, dacc.at[idx.at[par, b, 1]],
                                          ssem).wait()

        ip.wait()
        load_group(1)
        gathers(0)
        scatters(0)
        drain_scatters(0)

        def step(g, carry):
            par = g % 2
            pltpu.make_async_copy(idx_hbm.at[gbase + g], idx.at[par], isem).wait()
            gn = jnp.minimum(g + 1, nloc - 1)
            pltpu.async_copy(idx_hbm.at[gbase + gn], idx.at[(g + 1) % 2], isem)
            gathers(par)
            scatters(par)
            drain_scatters(par)
            return carry

        lax.fori_loop(1, nloc, step, 0)
        pltpu.make_async_copy(idx_hbm.at[gbase + nloc - 1],
                              idx.at[nloc % 2], isem).wait()
        plsc.subcore_barrier()

        wds = []
        for k in range(ROWS_PER_SUB // CHUNK):
            r0 = s * ROWS_PER_SUB + k * CHUNK
            wds.append(pltpu.async_copy(acc.at[pl.ds(r0, CHUNK)],
                                        agg_hbm.at[c, pl.ds(r0, CHUNK)], isem))
        if want_deg:
            wds.append(pltpu.async_copy(dacc.at[pl.ds(s * ROWS_PER_SUB, ROWS_PER_SUB)],
                                        deg_hbm.at[c, pl.ds(s * ROWS_PER_SUB,
                                                            ROWS_PER_SUB)], isem))
        for wd in wds:
            wd.wait()

    out_type = [jax.ShapeDtypeStruct((NC, N_PAD, d), jnp.float32)]
    if want_deg:
        out_type.append(jax.ShapeDtypeStruct((NC, N_PAD), jnp.float32))
    return pl.kernel(
        body,
        out_type=tuple(out_type),
        mesh=mesh,
        scratch_types=[
            pltpu.VMEM((2, GROUP, 2, CHUNK), jnp.int32),
            pltpu.VMEM((GROUP, CHUNK, d), jnp.float32),
            pltpu.VMEM((CHUNK,), jnp.float32),
            pltpu.VMEM_SHARED((N_PAD, d), jnp.float32),
            pltpu.VMEM_SHARED((N_PAD,), jnp.float32),
            pltpu.SemaphoreType.DMA,
            pltpu.SemaphoreType.DMA,
            pltpu.SemaphoreType.DMA,
        ],
    )


RB = 400


def _dense(d_in, d_out, relu):
    def body(aggp_ref, degp_ref, x_ref, wlt_ref, wrt_ref, bl_ref, o_ref):
        agg = aggp_ref[0] + aggp_ref[1]
        deg = degp_ref[0, 0, 0] + degp_ref[1, 0, 0]
        mean = agg / jnp.clip(deg, 1.0, None)[:, None]
        out = (jnp.dot(mean, wlt_ref[...], preferred_element_type=jnp.float32)
               + jnp.dot(x_ref[...], wrt_ref[...], preferred_element_type=jnp.float32)
               + bl_ref[...])
        o_ref[...] = jnp.maximum(out, 0.0) if relu else out

    return pl.pallas_call(
        body,
        grid=(N // RB,),
        in_specs=[
            pl.BlockSpec((NC, RB, d_in), lambda i: (0, i, 0)),
            pl.BlockSpec((NC, 1, 1, RB), lambda i: (0, i, 0, 0)),
            pl.BlockSpec((RB, d_in), lambda i: (i, 0)),
            pl.BlockSpec((d_in, d_out), lambda i: (0, 0)),
            pl.BlockSpec((d_in, d_out), lambda i: (0, 0)),
            pl.BlockSpec((1, d_out), lambda i: (0, 0)),
        ],
        out_specs=pl.BlockSpec((RB, d_out), lambda i: (i, 0)),
        out_shape=jax.ShapeDtypeStruct((N, d_out), jnp.float32),
    )


_segsum_deg = _sc_segsum(D_IN, want_deg=True)
_segsum_nodeg = _sc_segsum(D_IN, want_deg=False)
_dense1 = _dense(D_IN, D_HID, relu=True)
_dense2 = _dense(D_HID, D_OUT, relu=False)


@jax.jit
def kernel(x, edge_index, Wl1, bl1, Wr1, Wl2, bl2, Wr2):
    src = edge_index[0].astype(jnp.int32)
    dst = edge_index[1].astype(jnp.int32)
    pad = E_PAD - E
    src_p = jnp.concatenate([src, jnp.zeros((pad,), jnp.int32)]).reshape(-1, CHUNK)
    dst_p = jnp.concatenate([dst, jnp.full((pad,), N, jnp.int32)]).reshape(-1, CHUNK)
    idx_all = jnp.stack([src_p, dst_p], axis=1).reshape(-1, GROUP, 2, CHUNK)

    agg1, deg = _segsum_deg(x, idx_all)
    deg_r = deg[:, :N].reshape(NC, N // RB, 1, RB)
    h = _dense1(agg1, deg_r, x, Wl1.T, Wr1.T, bl1.reshape(1, -1))
    res2 = _segsum_nodeg(h, idx_all)
    agg2 = res2[0] if isinstance(res2, (tuple, list)) else res2
    return _dense2(agg2, deg_r, h, Wl2.T, Wr2.T, bl2.reshape(1, -1))


# speedup vs baseline: 1.4039x; 1.0991x over previous
import jax
import jax.numpy as jnp
from jax import lax
from jax.experimental import pallas as pl
from jax.experimental.pallas import tpu as pltpu
from jax.experimental.pallas import tpu_sc as plsc

N = 10000
E = 320000
D_IN = 128
D_HID = 128
D_OUT = 64

NC, NS = 2, 16
NW = NC * NS
CHUNK = 128
CHUNKS_PER_TILE = 80
GROUP = 2
NGROUPS = CHUNKS_PER_TILE // GROUP
P0, P1 = 77, 3
E_TILE = CHUNKS_PER_TILE * CHUNK
E_PAD = NW * E_TILE
N_PAD = 10240
ROWS_PER_SUB = N_PAD // NS


def _sc_segsum(d, want_deg):
    mesh = plsc.VectorSubcoreMesh(core_axis_name="c", subcore_axis_name="s")

    def body(x_hbm, idx_hbm, *rest):
        if want_deg:
            agg_hbm, deg_hbm = rest[0], rest[1]
            scr = rest[2:]
        else:
            agg_hbm = rest[0]
            scr = rest[1:]
        idx, rows, ones, acc, dacc, isem, gsem, ssem = scr
        c = lax.axis_index("c")
        s = lax.axis_index("s")
        nloc = jnp.where(c == 0, P0, P1)
        gbase = jnp.where(c == 0, s * P0, NS * P0 + s * P1)

        def load_group(g):
            return pltpu.async_copy(idx_hbm.at[gbase + g], idx.at[g % 2], isem)

        ip = load_group(0)

        def zr(i, _):
            for jj in range(d // 16):
                rows[0, i, pl.ds(jj * 16, 16)] = jnp.zeros((16,), jnp.float32)
            return 0
        lax.fori_loop(0, CHUNK, zr, 0)
        for jj in range(CHUNK // 16):
            ones[pl.ds(jj * 16, 16)] = jnp.ones((16,), jnp.float32)

        def zc(k, _):
            r0 = s * ROWS_PER_SUB + k * CHUNK
            pltpu.sync_copy(rows.at[0], acc.at[pl.ds(r0, CHUNK)])
            if want_deg:
                pltpu.sync_copy(rows.at[0, 0], dacc.at[pl.ds(r0, CHUNK)])
            return 0
        lax.fori_loop(0, ROWS_PER_SUB // CHUNK, zc, 0)
        plsc.subcore_barrier()

        def gathers(par):
            gds = [pltpu.async_copy(x_hbm.at[idx.at[par, b, 0]],
                                    rows.at[b], gsem)
                   for b in range(GROUP)]
            for gd in gds:
                gd.wait()

        def scatters(par):
            for b in range(GROUP):
                pltpu.async_copy(rows.at[b], acc.at[idx.at[par, b, 1]],
                                 ssem, add=True)
                if want_deg:
                    pltpu.async_copy(ones, dacc.at[idx.at[par, b, 1]],
                                     ssem, add=True)

        def drain_scatters(par):
            for b in range(GROUP):
                pltpu.make_async_copy(rows.at[b], acc.at[idx.at[par, b, 1]],
                                      ssem).wait()
                if want_deg:
                    pltpu.make_async_copy(ones, dacc.at[idx.at[par, b, 1]],
                                          ssem).wait()

        ip.wait()
        load_group(1)
        gathers(0)
        scatters(0)
        drain_scatters(0)

        def step(g, carry):
            par = g % 2
            pltpu.make_async_copy(idx_hbm.at[gbase + g], idx.at[par], isem).wait()
            gn = jnp.minimum(g + 1, nloc - 1)
            pltpu.async_copy(idx_hbm.at[gbase + gn], idx.at[(g + 1) % 2], isem)
            gathers(par)
            scatters(par)
            drain_scatters(par)
            return carry

        lax.fori_loop(1, nloc, step, 0)
        pltpu.make_async_copy(idx_hbm.at[gbase + nloc - 1],
                              idx.at[nloc % 2], isem).wait()
        plsc.subcore_barrier()

        wds = []
        for k in range(ROWS_PER_SUB // CHUNK):
            r0 = s * ROWS_PER_SUB + k * CHUNK
            wds.append(pltpu.async_copy(acc.at[pl.ds(r0, CHUNK)],
                                        agg_hbm.at[c, pl.ds(r0, CHUNK)], isem))
        if want_deg:
            wds.append(pltpu.async_copy(dacc.at[pl.ds(s * ROWS_PER_SUB, ROWS_PER_SUB)],
                                        deg_hbm.at[c, pl.ds(s * ROWS_PER_SUB,
                                                            ROWS_PER_SUB)], isem))
        for wd in wds:
            wd.wait()

    out_type = [jax.ShapeDtypeStruct((NC, N_PAD, d), jnp.float32)]
    if want_deg:
        out_type.append(jax.ShapeDtypeStruct((NC, N_PAD), jnp.float32))
    return pl.kernel(
        body,
        out_type=tuple(out_type),
        mesh=mesh,
        scratch_types=[
            pltpu.VMEM((2, GROUP, 2, CHUNK), jnp.int32),
            pltpu.VMEM((GROUP, CHUNK, d), jnp.float32),
            pltpu.VMEM((CHUNK,), jnp.float32),
            pltpu.VMEM_SHARED((N_PAD, d), jnp.float32),
            pltpu.VMEM_SHARED((N_PAD,), jnp.float32),
            pltpu.SemaphoreType.DMA,
            pltpu.SemaphoreType.DMA,
            pltpu.SemaphoreType.DMA,
        ],
    )


RB = 400


def _dense(d_in, d_out, relu):
    def body(aggp_ref, degp_ref, x_ref, wlt_ref, wrt_ref, bl_ref, o_ref):
        agg = aggp_ref[0] + aggp_ref[1]
        deg = degp_ref[0, 0, 0] + degp_ref[1, 0, 0]
        mean = agg / jnp.clip(deg, 1.0, None)[:, None]
        out = (jnp.dot(mean, wlt_ref[...], preferred_element_type=jnp.float32)
               + jnp.dot(x_ref[...], wrt_ref[...], preferred_element_type=jnp.float32)
               + bl_ref[...])
        o_ref[...] = jnp.maximum(out, 0.0) if relu else out

    return pl.pallas_call(
        body,
        grid=(N // RB,),
        in_specs=[
            pl.BlockSpec((NC, RB, d_in), lambda i: (0, i, 0)),
            pl.BlockSpec((NC, 1, 1, RB), lambda i: (0, i, 0, 0)),
            pl.BlockSpec((RB, d_in), lambda i: (i, 0)),
            pl.BlockSpec((d_in, d_out), lambda i: (0, 0)),
            pl.BlockSpec((d_in, d_out), lambda i: (0, 0)),
            pl.BlockSpec((1, d_out), lambda i: (0, 0)),
        ],
        out_specs=pl.BlockSpec((RB, d_out), lambda i: (i, 0)),
        out_shape=jax.ShapeDtypeStruct((N, d_out), jnp.float32),
    )


_segsum_deg = _sc_segsum(D_IN, want_deg=True)
_segsum_nodeg = _sc_segsum(D_IN, want_deg=False)
_dense1 = _dense(D_IN, D_HID, relu=True)
_dense2 = _dense(D_HID, D_OUT, relu=False)


@jax.jit
def kernel(x, edge_index, Wl1, bl1, Wr1, Wl2, bl2, Wr2):
    src = edge_index[0].astype(jnp.int32)
    dst = edge_index[1].astype(jnp.int32)
    pad = E_PAD - E
    src_p = jnp.concatenate([src, jnp.zeros((pad,), jnp.int32)]).reshape(-1, CHUNK)
    dst_p = jnp.concatenate([dst, jnp.full((pad,), N, jnp.int32)]).reshape(-1, CHUNK)
    idx_all = jnp.stack([src_p, dst_p], axis=1).reshape(-1, GROUP, 2, CHUNK)

    agg1, deg = _segsum_deg(x, idx_all)
    deg_r = deg[:, :N].reshape(NC, N // RB, 1, RB)
    h = _dense1(agg1, deg_r, x, Wl1.T, Wr1.T, bl1.reshape(1, -1))
    res2 = _segsum_nodeg(h, idx_all)
    agg2 = res2[0] if isinstance(res2, (tuple, list)) else res2
    return _dense2(agg2, deg_r, h, Wl2.T, Wr2.T, bl2.reshape(1, -1))
